# Initial kernel scaffold; baseline (speedup 1.0000x reference)
import jax
import jax.numpy as jnp
from jax.experimental import pallas as pl


def kernel(x, param_0, param_1, param_2, param_3, param_4, param_5, param_6):
    raise NotImplementedError("write your pallas kernel here")


# speedup vs baseline: 79.9331x; 79.9331x over previous
import jax
import jax.numpy as jnp
from jax import lax
from jax.experimental import pallas as pl
from jax.experimental.pallas import tpu as pltpu
from jax.experimental.pallas import tpu_sc as plsc

_RES = [720, 360, 180, 90, 45, 22, 11]
_BIG = (0, 1, 2)
_SMALL = (3, 4, 5, 6)
_NC = 2
_NS = 16
_NW = _NC * _NS
_B = 128
_NV = _B // 16
_CHUNK = 128
_NBLK = 248
_WPTS = _B * _NBLK
_NPAD = _WPTS * _NW
_NROW = _NPAD // _CHUNK
_FOUT = 2 * len(_RES)
_TCB = 8


def _level_math(latf, lonf, r):
    rows = r
    cols = 2 * r
    i0 = latf.astype(jnp.int32)
    j0 = lonf.astype(jnp.int32)
    ii0 = jnp.minimum(i0, rows)
    ii1 = jnp.minimum(i0 + 1, rows)
    jj0 = jnp.minimum(j0, cols)
    jj1 = jnp.minimum(j0 + 1, cols)
    wi0 = 1.0 - jnp.abs(latf - ii0.astype(jnp.float32))
    wi1 = 1.0 - jnp.abs(latf - ii1.astype(jnp.float32))
    wj0 = 1.0 - jnp.abs(lonf - jj0.astype(jnp.float32))
    wj1 = 1.0 - jnp.abs(lonf - jj1.astype(jnp.float32))
    jj0w = jnp.where(jj0 == cols, 0, jj0)
    jj1w = jnp.where(jj1 == cols, 0, jj1)
    pole0 = (ii0 == 0) | (ii0 == rows)
    pole1 = (ii1 == 0) | (ii1 == rows)
    j00 = jnp.where(pole0, 0, jj0w)
    j01 = jnp.where(pole0, 0, jj1w)
    j10 = jnp.where(pole1, 0, jj0w)
    j11 = jnp.where(pole1, 0, jj1w)
    c = cols + 1
    b0 = ii0 * c
    b1 = ii1 * c
    f = (b0 + j00, b0 + j01, b1 + j10, b1 + j11)
    w = (wi0 * wj0, wi0 * wj1, wi1 * wj0, wi1 * wj1)
    return f, w


def _tc_index_kernel(lat_ref, lon_ref, idx_ref):
    lat = lat_ref[...]
    lon = lon_ref[...]
    ulat = (lat + 90.0) / 180.0
    ulon = (lon / 360.0) * 2.0
    for li, l in enumerate(_BIG):
        r = _RES[l]
        fs, _ = _level_math(ulat * r, ulon * r, r)
        for n in range(4):
            idx_ref[li * 4 + n] = fs[n]


def _sc_kernel(lat_h, lon_h, idx_h, t0, t1, t2, s3, s4, s5, s6, out_h,
               *scratch):
    nchunk = _B // _CHUNK
    idx2d = scratch[0]
    lat_v, lon_v = scratch[1], scratch[2]
    small_v = scratch[3:7]
    rows_refs = scratch[7:10]
    out_v = scratch[10]
    sem = scratch[11]
    wid = lax.axis_index("s") * _NC + lax.axis_index("c")
    lane = lax.iota(jnp.int32, 16)

    pltpu.sync_copy(s3, small_v[0])
    pltpu.sync_copy(s4, small_v[1])
    pltpu.sync_copy(s5, small_v[2])
    pltpu.sync_copy(s6, small_v[3])

    tbl_hbm = (t0, t1, t2)

    def block_body(b, carry):
        base = wid * _WPTS + b * _B
        brow = base // _CHUNK
        cps = [pltpu.async_copy(lat_h.at[pl.ds(base, _B)], lat_v, sem),
               pltpu.async_copy(lon_h.at[pl.ds(base, _B)], lon_v, sem)]
        for rrow in range(12):
            for ch in range(nchunk):
                cps.append(pltpu.async_copy(
                    idx_h.at[rrow, brow + ch],
                    idx2d.at[rrow * nchunk + ch], sem))
        for cp in cps:
            cp.wait()

        copies = []
        for li in range(3):
            for n in range(4):
                for ch in range(nchunk):
                    j = (li * 4 + n) * nchunk + ch
                    copies.append(pltpu.async_copy(
                        tbl_hbm[li].at[idx2d.at[4 * j]],
                        rows_refs[li].at[pl.ds(n * _B + ch * _CHUNK, _CHUNK)],
                        sem))

        def pass2(v, carry2):
            lat16 = lat_v[pl.ds(v * 16, 16)]
            lon16 = lon_v[pl.ds(v * 16, 16)]
            ulat = (lat16 + 90.0) / 180.0
            ulon = (lon16 / 360.0) * 2.0
            prow = v * 16 + lane
            for si, l in enumerate(_SMALL):
                r = _RES[l]
                fs, ws = _level_math(ulat * r, ulon * r, r)
                tv = small_v[si]
                for c in range(2):
                    acc = ws[0] * plsc.load_gather(tv, [fs[0] * 2 + c])
                    acc = acc + ws[1] * plsc.load_gather(tv, [fs[1] * 2 + c])
                    acc = acc + ws[2] * plsc.load_gather(tv, [fs[2] * 2 + c])
                    acc = acc + ws[3] * plsc.load_gather(tv, [fs[3] * 2 + c])
                    cc = jnp.full((16,), 2 * l + c, jnp.int32)
                    plsc.store_scatter(out_v, [prow, cc], acc)
            return carry2

        lax.fori_loop(0, _NV, pass2, 0)

        for cp in copies:
            cp.wait()

        def pass3(v, carry3):
            lat16 = lat_v[pl.ds(v * 16, 16)]
            lon16 = lon_v[pl.ds(v * 16, 16)]
            ulat = (lat16 + 90.0) / 180.0
            ulon = (lon16 / 360.0) * 2.0
            prow = v * 16 + lane
            for li, l in enumerate(_BIG):
                r = _RES[l]
                _, ws = _level_math(ulat * r, ulon * r, r)
                rv = rows_refs[li]
                for c in range(2):
                    cc = jnp.full((16,), c, jnp.int32)
                    acc = jnp.zeros((16,), jnp.float32)
                    for n in range(4):
                        g = plsc.load_gather(rv, [n * _B + prow, cc])
                        acc = acc + ws[n] * g
                    oc = jnp.full((16,), 2 * l + c, jnp.int32)
                    plsc.store_scatter(out_v, [prow, oc], acc)
            return carry3

        lax.fori_loop(0, _NV, pass3, 0)

        pltpu.sync_copy(out_v, out_h.at[pl.ds(base, _B)])
        return carry

    lax.fori_loop(0, _NBLK, block_body, 0)


def kernel(x, param_0, param_1, param_2, param_3, param_4, param_5, param_6):
    n = x.shape[0]
    lat = jnp.pad(x[:, 0], (0, _NPAD - n))
    lon = jnp.pad(x[:, 1], (0, _NPAD - n))
    params = [param_0, param_1, param_2, param_3, param_4, param_5, param_6]
    big = [params[l].reshape(-1, 2) for l in _BIG]
    small = [params[l].reshape(-1) for l in _SMALL]

    idx_all = pl.pallas_call(
        _tc_index_kernel,
        grid=(_NROW // _TCB,),
        in_specs=[pl.BlockSpec((_TCB, _CHUNK), lambda i: (i, 0)),
                  pl.BlockSpec((_TCB, _CHUNK), lambda i: (i, 0))],
        out_specs=pl.BlockSpec((12, _TCB, _CHUNK), lambda i: (0, i, 0)),
        out_shape=jax.ShapeDtypeStruct((12, _NROW, _CHUNK), jnp.int32),
    )(lat.reshape(_NROW, _CHUNK), lon.reshape(_NROW, _CHUNK))

    mesh = plsc.VectorSubcoreMesh(core_axis_name="c", subcore_axis_name="s")
    run = pl.kernel(
        _sc_kernel,
        out_type=jax.ShapeDtypeStruct((_NPAD, _FOUT), jnp.float32),
        mesh=mesh,
        compiler_params=pltpu.CompilerParams(
            needs_layout_passes=False, use_tc_tiling_on_sc=False),
        scratch_types=(
            [pltpu.VMEM((4 * 12 * (_B // _CHUNK), _CHUNK), jnp.int32),
             pltpu.VMEM((_B,), jnp.float32),
             pltpu.VMEM((_B,), jnp.float32)]
            + [pltpu.VMEM((s.shape[0],), jnp.float32) for s in small]
            + [pltpu.VMEM((2048, 2), jnp.float32)
               for _ in range(3)]
            + [pltpu.VMEM((_B, _FOUT), jnp.float32),
               pltpu.SemaphoreType.DMA]
        ),
    )
    out = run(lat, lon, idx_all, big[0], big[1], big[2],
              small[0], small[1], small[2], small[3])
    return out[:n]
